# baseline (speedup 1.0000x reference)
import jax
import jax.numpy as jnp
from jax.experimental import pallas as pl
from jax.experimental.pallas import tpu as pltpu

_BLK = 1024


def _gate_block(x_ref, wt_ref, b_ref, o_ref):
    logits = jnp.dot(x_ref[...], wt_ref[...],
                     preferred_element_type=jnp.float32) + b_ref[...]
    l0 = logits[:, 0:1]
    lrest = jnp.max(logits[:, 1:], axis=1, keepdims=True)
    is0 = (l0 >= lrest).astype(jnp.float32)
    o_ref[...] = jnp.concatenate([is0, 1.0 - is0], axis=1)


@jax.jit
def kernel(x, W, b):
    B, S, D = x.shape
    K = W.shape[0]
    M = B * S
    x2 = x.reshape(M, D)
    wt = W.T
    b2 = b.reshape(1, K)
    out = pl.pallas_call(
        _gate_block,
        grid=(M // _BLK,),
        in_specs=[
            pl.BlockSpec((_BLK, D), lambda i: (i, 0)),
            pl.BlockSpec((D, K), lambda i: (0, 0)),
            pl.BlockSpec((1, K), lambda i: (0, 0)),
        ],
        out_specs=pl.BlockSpec((_BLK, 2), lambda i: (i, 0)),
        out_shape=jax.ShapeDtypeStruct((M, 2), jnp.float32),
        compiler_params=pltpu.CompilerParams(
            dimension_semantics=("parallel",),
        ),
    )(x2, wt, b2)
    return out.reshape(B, S, 2)


# speedup vs baseline: 1.0103x; 1.0103x over previous
import jax
import jax.numpy as jnp
from jax.experimental import pallas as pl
from jax.experimental.pallas import tpu as pltpu

_BLK = 256
_NSTREAM = 4


def _gate_block(*refs):
    x_refs = refs[:_NSTREAM]
    wt_ref, b_ref = refs[_NSTREAM], refs[_NSTREAM + 1]
    o_refs = refs[_NSTREAM + 2:]
    for x_ref, o_ref in zip(x_refs, o_refs):
        logits = jnp.dot(x_ref[...], wt_ref[...],
                         preferred_element_type=jnp.float32) + b_ref[...]
        l0 = logits[:, 0:1]
        lrest = jnp.max(logits[:, 1:], axis=1, keepdims=True)
        is0 = (l0 >= lrest).astype(jnp.float32)
        o_ref[...] = jnp.concatenate([is0, 1.0 - is0], axis=1)


@jax.jit
def kernel(x, W, b):
    B, S, D = x.shape
    K = W.shape[0]
    M = B * S
    H = M // _NSTREAM
    nb = H // _BLK
    x2 = x.reshape(M, D)
    wt = W.T
    b2 = b.reshape(1, K)

    def make_in(s):
        return pl.BlockSpec((_BLK, D), lambda i, s=s: (i + s * nb, 0))

    outs = pl.pallas_call(
        _gate_block,
        grid=(nb,),
        in_specs=[make_in(s) for s in range(_NSTREAM)] + [
            pl.BlockSpec((D, K), lambda i: (0, 0)),
            pl.BlockSpec((1, K), lambda i: (0, 0)),
        ],
        out_specs=[
            pl.BlockSpec((_BLK, 2), lambda i: (i, 0))
            for _ in range(_NSTREAM)
        ],
        out_shape=[
            jax.ShapeDtypeStruct((H, 2), jnp.float32)
            for _ in range(_NSTREAM)
        ],
        compiler_params=pltpu.CompilerParams(
            dimension_semantics=("arbitrary",),
        ),
    )(*([x2] * _NSTREAM), wt, b2)
    return jnp.concatenate(outs, axis=0).reshape(B, S, 2)


# speedup vs baseline: 1.0196x; 1.0092x over previous
import jax
import jax.numpy as jnp
from jax.experimental import pallas as pl
from jax.experimental.pallas import tpu as pltpu

_BLK = 512
_NSTREAM = 2


def _gate_block(*refs):
    x_refs = refs[:_NSTREAM]
    wt_ref, b_ref = refs[_NSTREAM], refs[_NSTREAM + 1]
    o_refs = refs[_NSTREAM + 2:]
    for x_ref, o_ref in zip(x_refs, o_refs):
        logits = jnp.dot(x_ref[...], wt_ref[...],
                         preferred_element_type=jnp.float32) + b_ref[...]
        l0 = logits[:, 0:1]
        lrest = jnp.max(logits[:, 1:], axis=1, keepdims=True)
        is0 = (l0 >= lrest).astype(jnp.float32)
        o_ref[...] = jnp.concatenate([is0, 1.0 - is0], axis=1)


@jax.jit
def kernel(x, W, b):
    B, S, D = x.shape
    K = W.shape[0]
    M = B * S
    H = M // _NSTREAM
    nb = H // _BLK
    x2 = x.reshape(M, D)
    wt = W.T
    b2 = b.reshape(1, K)

    def make_in(s):
        return pl.BlockSpec((_BLK, D), lambda i, s=s: (i + s * nb, 0))

    outs = pl.pallas_call(
        _gate_block,
        grid=(nb,),
        in_specs=[make_in(s) for s in range(_NSTREAM)] + [
            pl.BlockSpec((D, K), lambda i: (0, 0)),
            pl.BlockSpec((1, K), lambda i: (0, 0)),
        ],
        out_specs=[
            pl.BlockSpec((_BLK, 2), lambda i: (i, 0))
            for _ in range(_NSTREAM)
        ],
        out_shape=[
            jax.ShapeDtypeStruct((H, 2), jnp.float32)
            for _ in range(_NSTREAM)
        ],
        compiler_params=pltpu.CompilerParams(
            dimension_semantics=("arbitrary",),
        ),
    )(*([x2] * _NSTREAM), wt, b2)
    return jnp.concatenate(outs, axis=0).reshape(B, S, 2)
